# Initial kernel scaffold; baseline (speedup 1.0000x reference)
import functools

import jax
import jax.numpy as jnp
from jax import lax
from jax.experimental import pallas as pl
from jax.experimental.pallas import tpu as pltpu
from jax.experimental.pallas import tpu_sc as plsc

N = 10000
NPAD = 10240
E = 160000
EPAD = 163840
CH = 128
NROWS = EPAD // CH
NC, NS = 2, 16
NW = NC * NS
NCHW = NROWS // NW
RPT = NPAD // NS

_mesh = plsc.VectorSubcoreMesh(core_axis_name="c", subcore_axis_name="s")


@functools.partial(
    pl.kernel,
    out_type=jax.ShapeDtypeStruct((NC, NPAD, 16), jnp.float32),
    mesh=_mesh,
    scratch_types=[
        pltpu.VMEM((NCHW, CH), jnp.int32),
        pltpu.VMEM((CH, 16), jnp.float32),
        pltpu.VMEM_SHARED((NPAD, 16), jnp.float32),
    ],
)
def _deg_kernel(dst_hbm, ones_hbm, zeros_hbm, out_hbm, didx, ones_v, acc):
    c = lax.axis_index("c")
    s = lax.axis_index("s")
    g = c * NS + s
    pltpu.sync_copy(zeros_hbm.at[pl.ds(s * RPT, RPT)], acc.at[pl.ds(s * RPT, RPT)])
    pltpu.sync_copy(ones_hbm, ones_v)
    pltpu.sync_copy(dst_hbm.at[pl.ds(g * NCHW, NCHW)], didx)
    plsc.subcore_barrier()

    def body(j, carry):
        pltpu.sync_copy(ones_v, acc.at[didx.at[j]], add=True)
        return carry

    lax.fori_loop(0, NCHW, body, 0)
    plsc.subcore_barrier()
    pltpu.sync_copy(acc.at[pl.ds(s * RPT, RPT)], out_hbm.at[c, pl.ds(s * RPT, RPT)])


@functools.partial(
    pl.kernel,
    out_type=jax.ShapeDtypeStruct((NC, NPAD, 128), jnp.float32),
    mesh=_mesh,
    scratch_types=[
        pltpu.VMEM((NCHW, CH), jnp.int32),
        pltpu.VMEM((NCHW, CH), jnp.int32),
        pltpu.VMEM((CH, 128), jnp.float32),
        pltpu.VMEM_SHARED((NPAD, 128), jnp.float32),
        pltpu.SemaphoreType.DMA,
    ],
)
def _agg_kernel(src_hbm, dst_hbm, table_hbm, zeros_hbm, out_hbm,
                sidx, didx, buf, acc, gsem):
    c = lax.axis_index("c")
    s = lax.axis_index("s")
    g = c * NS + s
    pltpu.sync_copy(zeros_hbm.at[pl.ds(s * RPT, RPT)], acc.at[pl.ds(s * RPT, RPT)])
    pltpu.sync_copy(src_hbm.at[pl.ds(g * NCHW, NCHW)], sidx)
    pltpu.sync_copy(dst_hbm.at[pl.ds(g * NCHW, NCHW)], didx)
    plsc.subcore_barrier()

    def body(j, carry):
        pltpu.async_copy(table_hbm.at[sidx.at[j]], buf, gsem).wait()
        pltpu.sync_copy(buf, acc.at[didx.at[j]], add=True)
        return carry

    lax.fori_loop(0, NCHW, body, 0)
    plsc.subcore_barrier()
    pltpu.sync_copy(acc.at[pl.ds(s * RPT, RPT)], out_hbm.at[c, pl.ds(s * RPT, RPT)])


BR = 512
GRID = NPAD // BR


def _dvec(deg16):
    deg = deg16[0, :, 0] + deg16[1, :, 0] + 1.0
    return lax.rsqrt(deg)[:, None]


def _scale_body(deg16_ref, x_ref, y0_ref, y1_ref):
    d = _dvec(deg16_ref[...])
    y = x_ref[...] * d
    y0_ref[...] = y[:, :128]
    y1_ref[...] = y[:, 128:]


_scale_kernel = pl.pallas_call(
    _scale_body,
    grid=(GRID,),
    in_specs=[
        pl.BlockSpec((NC, BR, 16), lambda i: (0, i, 0)),
        pl.BlockSpec((BR, 256), lambda i: (i, 0)),
    ],
    out_specs=[
        pl.BlockSpec((BR, 128), lambda i: (i, 0)),
        pl.BlockSpec((BR, 128), lambda i: (i, 0)),
    ],
    out_shape=[
        jax.ShapeDtypeStruct((NPAD, 128), jnp.float32),
        jax.ShapeDtypeStruct((NPAD, 128), jnp.float32),
    ],
)


def _layer_body(s10_ref, s11_ref, deg16_ref, x_ref, w1_ref, b1_ref, w2_ref,
                y2_ref, z2_ref):
    d = _dvec(deg16_ref[...])
    s1 = jnp.concatenate(
        [s10_ref[0] + s10_ref[1], s11_ref[0] + s11_ref[1]], axis=1)
    z1 = d * s1 + (d * d) * x_ref[...]
    h = jnp.maximum(
        jnp.dot(z1, w1_ref[...], preferred_element_type=jnp.float32)
        + b1_ref[...], 0.0)
    z2 = jnp.dot(h, w2_ref[...], preferred_element_type=jnp.float32)
    z2_ref[...] = z2
    y2_ref[...] = d * z2


_layer_kernel = pl.pallas_call(
    _layer_body,
    grid=(GRID,),
    in_specs=[
        pl.BlockSpec((NC, BR, 128), lambda i: (0, i, 0)),
        pl.BlockSpec((NC, BR, 128), lambda i: (0, i, 0)),
        pl.BlockSpec((NC, BR, 16), lambda i: (0, i, 0)),
        pl.BlockSpec((BR, 256), lambda i: (i, 0)),
        pl.BlockSpec((256, 512), lambda i: (0, 0)),
        pl.BlockSpec((1, 512), lambda i: (0, 0)),
        pl.BlockSpec((512, 128), lambda i: (0, 0)),
    ],
    out_specs=[
        pl.BlockSpec((BR, 128), lambda i: (i, 0)),
        pl.BlockSpec((BR, 128), lambda i: (i, 0)),
    ],
    out_shape=[
        jax.ShapeDtypeStruct((NPAD, 128), jnp.float32),
        jax.ShapeDtypeStruct((NPAD, 128), jnp.float32),
    ],
)


def _out_body(s2_ref, deg16_ref, z2_ref, b2_ref, out_ref):
    d = _dvec(deg16_ref[...])
    out_ref[...] = (d * (s2_ref[0] + s2_ref[1])
                    + (d * d) * z2_ref[...] + b2_ref[...])


_out_kernel = pl.pallas_call(
    _out_body,
    grid=(GRID,),
    in_specs=[
        pl.BlockSpec((NC, BR, 128), lambda i: (0, i, 0)),
        pl.BlockSpec((NC, BR, 16), lambda i: (0, i, 0)),
        pl.BlockSpec((BR, 128), lambda i: (i, 0)),
        pl.BlockSpec((1, 128), lambda i: (0, 0)),
    ],
    out_specs=pl.BlockSpec((BR, 128), lambda i: (i, 0)),
    out_shape=jax.ShapeDtypeStruct((NPAD, 128), jnp.float32),
)


def kernel(x, edge_index, W1, b1, W2, b2):
    src = edge_index[0].astype(jnp.int32)
    dst = edge_index[1].astype(jnp.int32)
    fill = jnp.full((EPAD - E,), N, jnp.int32)
    src_p = jnp.concatenate([src, fill]).reshape(NROWS, CH)
    dst_p = jnp.concatenate([dst, fill]).reshape(NROWS, CH)
    x_p = jnp.zeros((NPAD, 256), jnp.float32).at[:N].set(x)
    zeros128 = jnp.zeros((NPAD, 128), jnp.float32)
    zeros16 = jnp.zeros((NPAD, 16), jnp.float32)
    ones16 = jnp.ones((CH, 16), jnp.float32)

    deg16 = _deg_kernel(dst_p, ones16, zeros16)
    y0, y1 = _scale_kernel(deg16, x_p)
    s10 = _agg_kernel(src_p, dst_p, y0, zeros128)
    s11 = _agg_kernel(src_p, dst_p, y1, zeros128)
    y2, z2 = _layer_kernel(s10, s11, deg16, x_p, W1, b1.reshape(1, -1), W2)
    s2 = _agg_kernel(src_p, dst_p, y2, zeros128)
    out = _out_kernel(s2, deg16, z2, b2.reshape(1, -1))
    return out[:N]


# speedup vs baseline: 16.6735x; 16.6735x over previous
import jax
import jax.numpy as jnp
from jax import lax
from jax.experimental import pallas as pl
from jax.experimental.pallas import tpu as pltpu
from jax.experimental.pallas import tpu_sc as plsc

N = 10000
NPAD = 10240
E = 160000
EPAD = 163840
NC, NS = 2, 16
NW = NC * NS
RPT = NPAD // NS
DEGW = 128

CHD = 128
NCHD = EPAD // CHD // NW

CH = 64
NCHW = EPAD // CH // NW
NBUF = 4
NGRP = NCHW // NBUF

_mesh = plsc.VectorSubcoreMesh(core_axis_name="c", subcore_axis_name="s")


def _deg_body(dst_hbm, ones_hbm, zeros_hbm, out_hbm, didx, ones_v, acc):
    c = lax.axis_index("c")
    s = lax.axis_index("s")
    g = c * NS + s
    pltpu.sync_copy(zeros_hbm.at[pl.ds(s * RPT, RPT)], acc.at[pl.ds(s * RPT, RPT)])
    pltpu.sync_copy(ones_hbm, ones_v)
    pltpu.sync_copy(dst_hbm.at[pl.ds(g * NCHD, NCHD)], didx)
    plsc.subcore_barrier()

    def body(j, carry):
        pltpu.sync_copy(ones_v, acc.at[didx.at[j]], add=True)
        return carry

    lax.fori_loop(0, NCHD, body, 0)
    plsc.subcore_barrier()
    pltpu.sync_copy(acc.at[pl.ds(s * RPT, RPT)], out_hbm.at[c, pl.ds(s * RPT, RPT)])


def _make_deg(interpret=False, width=DEGW):
    return pl.kernel(
        _deg_body,
        out_type=jax.ShapeDtypeStruct((NC, NPAD, width), jnp.float32),
        mesh=_mesh,
        scratch_types=[
            pltpu.VMEM((NCHD, CHD), jnp.int32),
            pltpu.VMEM((CHD, width), jnp.float32),
            pltpu.VMEM_SHARED((NPAD, width), jnp.float32),
        ],
        interpret=interpret,
    )


_deg_kernel = _make_deg()


def _agg_body(src_hbm, dst_hbm, table_hbm, zeros_hbm, out_hbm,
              sidx, didx, bufs, acc, gsem, isem, *ssems):
    c = lax.axis_index("c")
    s = lax.axis_index("s")
    base = (c * NS + s) * NCHW
    pltpu.sync_copy(zeros_hbm.at[pl.ds(s * RPT, RPT)], acc.at[pl.ds(s * RPT, RPT)])
    pltpu.sync_copy(src_hbm.at[pl.ds(base, NBUF)], sidx.at[pl.ds(0, NBUF)])
    pltpu.sync_copy(dst_hbm.at[pl.ds(base, NBUF)], didx.at[pl.ds(0, NBUF)])
    plsc.subcore_barrier()

    for b in range(NBUF):
        pltpu.async_copy(table_hbm.at[sidx.at[b]], bufs.at[b], gsem)
    pltpu.async_copy(src_hbm.at[pl.ds(base + NBUF, NBUF)],
                     sidx.at[pl.ds(NBUF, NBUF)], isem)
    pltpu.async_copy(dst_hbm.at[pl.ds(base + NBUF, NBUF)],
                     didx.at[pl.ds(NBUF, NBUF)], isem)

    def pair_loop(k, carry):
        for gi, p in ((2 * k, 0), (2 * k + 1, 1)):
            for b in range(NBUF):
                pltpu.make_async_copy(table_hbm.at[sidx.at[p * NBUF + b]],
                                      bufs.at[b], gsem).wait()
            pltpu.make_async_copy(src_hbm.at[pl.ds(base, NBUF)],
                                  sidx.at[pl.ds((1 - p) * NBUF, NBUF)],
                                  isem).wait()
            pltpu.make_async_copy(dst_hbm.at[pl.ds(base, NBUF)],
                                  didx.at[pl.ds((1 - p) * NBUF, NBUF)],
                                  isem).wait()
            descs = []
            for b in range(NBUF):
                descs.append(pltpu.async_copy(
                    bufs.at[b], acc.at[didx.at[p * NBUF + b]], ssems[b],
                    add=True))
            for b in range(NBUF):
                descs[b].wait()
            nxt = gi + 2
            pltpu.async_copy(src_hbm.at[pl.ds(base + nxt * NBUF, NBUF)],
                             sidx.at[pl.ds(p * NBUF, NBUF)], isem)
            pltpu.async_copy(dst_hbm.at[pl.ds(base + nxt * NBUF, NBUF)],
                             didx.at[pl.ds(p * NBUF, NBUF)], isem)
            for b in range(NBUF):
                pltpu.async_copy(table_hbm.at[sidx.at[(1 - p) * NBUF + b]],
                                 bufs.at[b], gsem)
        return carry

    lax.fori_loop(0, NGRP // 2 - 1, pair_loop, 0)
    for gi, p, more in ((NGRP - 2, 0, True), (NGRP - 1, 1, False)):
        for b in range(NBUF):
            pltpu.make_async_copy(table_hbm.at[sidx.at[p * NBUF + b]],
                                  bufs.at[b], gsem).wait()
        if more:
            pltpu.make_async_copy(src_hbm.at[pl.ds(base, NBUF)],
                                  sidx.at[pl.ds((1 - p) * NBUF, NBUF)],
                                  isem).wait()
            pltpu.make_async_copy(dst_hbm.at[pl.ds(base, NBUF)],
                                  didx.at[pl.ds((1 - p) * NBUF, NBUF)],
                                  isem).wait()
        descs = []
        for b in range(NBUF):
            descs.append(pltpu.async_copy(
                bufs.at[b], acc.at[didx.at[p * NBUF + b]], ssems[b],
                add=True))
        for b in range(NBUF):
            descs[b].wait()
        if more:
            for b in range(NBUF):
                pltpu.async_copy(table_hbm.at[sidx.at[(1 - p) * NBUF + b]],
                                 bufs.at[b], gsem)
    plsc.subcore_barrier()
    pltpu.sync_copy(acc.at[pl.ds(s * RPT, RPT)], out_hbm.at[c, pl.ds(s * RPT, RPT)])


def _make_agg(interpret=False):
    return pl.kernel(
        _agg_body,
        out_type=jax.ShapeDtypeStruct((NC, NPAD, 128), jnp.float32),
        mesh=_mesh,
        scratch_types=[
            pltpu.VMEM((2 * NBUF, CH), jnp.int32),
            pltpu.VMEM((2 * NBUF, CH), jnp.int32),
            pltpu.VMEM((NBUF, CH, 128), jnp.float32),
            pltpu.VMEM_SHARED((NPAD, 128), jnp.float32),
        ] + [pltpu.SemaphoreType.DMA] * (2 + NBUF),
        interpret=interpret,
    )


_agg_kernel = _make_agg()


BR = 512
GRID = NPAD // BR


def _dvec(deg16):
    deg = deg16[0, :, 0] + deg16[1, :, 0] + 1.0
    return lax.rsqrt(deg)[:, None]


def _scale_body(deg16_ref, x_ref, y0_ref, y1_ref):
    d = _dvec(deg16_ref[...])
    y = x_ref[...] * d
    y0_ref[...] = y[:, :128]
    y1_ref[...] = y[:, 128:]


_scale_kernel = pl.pallas_call(
    _scale_body,
    grid=(GRID,),
    in_specs=[
        pl.BlockSpec((NC, BR, DEGW), lambda i: (0, i, 0)),
        pl.BlockSpec((BR, 256), lambda i: (i, 0)),
    ],
    out_specs=[
        pl.BlockSpec((BR, 128), lambda i: (i, 0)),
        pl.BlockSpec((BR, 128), lambda i: (i, 0)),
    ],
    out_shape=[
        jax.ShapeDtypeStruct((NPAD, 128), jnp.float32),
        jax.ShapeDtypeStruct((NPAD, 128), jnp.float32),
    ],
)


def _layer_body(s10_ref, s11_ref, deg16_ref, x_ref, w1_ref, b1_ref, w2_ref,
                y2_ref, z2_ref):
    d = _dvec(deg16_ref[...])
    s1 = jnp.concatenate(
        [s10_ref[0] + s10_ref[1], s11_ref[0] + s11_ref[1]], axis=1)
    z1 = d * s1 + (d * d) * x_ref[...]
    h = jnp.maximum(
        jnp.dot(z1, w1_ref[...], preferred_element_type=jnp.float32)
        + b1_ref[...], 0.0)
    z2 = jnp.dot(h, w2_ref[...], preferred_element_type=jnp.float32)
    z2_ref[...] = z2
    y2_ref[...] = d * z2


_layer_kernel = pl.pallas_call(
    _layer_body,
    grid=(GRID,),
    in_specs=[
        pl.BlockSpec((NC, BR, 128), lambda i: (0, i, 0)),
        pl.BlockSpec((NC, BR, 128), lambda i: (0, i, 0)),
        pl.BlockSpec((NC, BR, DEGW), lambda i: (0, i, 0)),
        pl.BlockSpec((BR, 256), lambda i: (i, 0)),
        pl.BlockSpec((256, 512), lambda i: (0, 0)),
        pl.BlockSpec((1, 512), lambda i: (0, 0)),
        pl.BlockSpec((512, 128), lambda i: (0, 0)),
    ],
    out_specs=[
        pl.BlockSpec((BR, 128), lambda i: (i, 0)),
        pl.BlockSpec((BR, 128), lambda i: (i, 0)),
    ],
    out_shape=[
        jax.ShapeDtypeStruct((NPAD, 128), jnp.float32),
        jax.ShapeDtypeStruct((NPAD, 128), jnp.float32),
    ],
)


def _out_body(s2_ref, deg16_ref, z2_ref, b2_ref, out_ref):
    d = _dvec(deg16_ref[...])
    out_ref[...] = (d * (s2_ref[0] + s2_ref[1])
                    + (d * d) * z2_ref[...] + b2_ref[...])


_out_kernel = pl.pallas_call(
    _out_body,
    grid=(GRID,),
    in_specs=[
        pl.BlockSpec((NC, BR, 128), lambda i: (0, i, 0)),
        pl.BlockSpec((NC, BR, DEGW), lambda i: (0, i, 0)),
        pl.BlockSpec((BR, 128), lambda i: (i, 0)),
        pl.BlockSpec((1, 128), lambda i: (0, 0)),
    ],
    out_specs=pl.BlockSpec((BR, 128), lambda i: (i, 0)),
    out_shape=jax.ShapeDtypeStruct((NPAD, 128), jnp.float32),
)


def kernel(x, edge_index, W1, b1, W2, b2):
    src = edge_index[0].astype(jnp.int32)
    dst = edge_index[1].astype(jnp.int32)
    fill_src = jnp.arange(EPAD - E, dtype=jnp.int32) % N
    fill_dst = jnp.full((EPAD - E,), N, jnp.int32)
    src_p = jnp.concatenate([src, fill_src]).reshape(EPAD // CH, CH)
    dst_p = jnp.concatenate([dst, fill_dst]).reshape(EPAD // CH, CH)
    x_p = jnp.zeros((NPAD, 256), jnp.float32).at[:N].set(x)
    zeros128 = jnp.zeros((NPAD, 128), jnp.float32)
    onesw = jnp.ones((CHD, DEGW), jnp.float32)

    deg16 = _deg_kernel(dst_p.reshape(EPAD // CHD, CHD), onesw, zeros128)
    y0, y1 = _scale_kernel(deg16, x_p)
    s10 = _agg_kernel(src_p, dst_p, y0, zeros128)
    s11 = _agg_kernel(src_p, dst_p, y1, zeros128)
    y2, z2 = _layer_kernel(s10, s11, deg16, x_p, W1, b1.reshape(1, -1), W2)
    s2 = _agg_kernel(src_p, dst_p, y2, zeros128)
    out = _out_kernel(s2, deg16, z2, b2.reshape(1, -1))
    return out[:N]


# speedup vs baseline: 17.1154x; 1.0265x over previous
import jax
import jax.numpy as jnp
from jax import lax
from jax.experimental import pallas as pl
from jax.experimental.pallas import tpu as pltpu
from jax.experimental.pallas import tpu_sc as plsc

N = 10000
NPAD = 10240
E = 160000
EPAD = 163840
NC, NS = 2, 16
NW = NC * NS
RPT = NPAD // NS
DEGW = 128

CHD = 128
NCHD = EPAD // CHD // NW

CH = 128
NCHW = EPAD // CH // NW
NBUF = 2
NGRP = NCHW // NBUF

_mesh = plsc.VectorSubcoreMesh(core_axis_name="c", subcore_axis_name="s")


def _deg_body(dst_hbm, ones_hbm, zeros_hbm, out_hbm, didx, ones_v, acc):
    c = lax.axis_index("c")
    s = lax.axis_index("s")
    g = c * NS + s
    pltpu.sync_copy(zeros_hbm.at[pl.ds(s * RPT, RPT)], acc.at[pl.ds(s * RPT, RPT)])
    pltpu.sync_copy(ones_hbm, ones_v)
    pltpu.sync_copy(dst_hbm.at[pl.ds(g * NCHD, NCHD)], didx)
    plsc.subcore_barrier()

    def body(j, carry):
        pltpu.sync_copy(ones_v, acc.at[didx.at[j]], add=True)
        return carry

    lax.fori_loop(0, NCHD, body, 0)
    plsc.subcore_barrier()
    pltpu.sync_copy(acc.at[pl.ds(s * RPT, RPT)], out_hbm.at[c, pl.ds(s * RPT, RPT)])


def _make_deg(interpret=False, width=DEGW):
    return pl.kernel(
        _deg_body,
        out_type=jax.ShapeDtypeStruct((NC, NPAD, width), jnp.float32),
        mesh=_mesh,
        scratch_types=[
            pltpu.VMEM((NCHD, CHD), jnp.int32),
            pltpu.VMEM((CHD, width), jnp.float32),
            pltpu.VMEM_SHARED((NPAD, width), jnp.float32),
        ],
        interpret=interpret,
    )


_deg_kernel = _make_deg()


def _agg_body(src_hbm, dst_hbm, table_hbm, zeros_hbm, out_hbm,
              sidx, didx, bufs, acc, *sems):
    gsems, ssems = sems[:NBUF], sems[NBUF:]
    c = lax.axis_index("c")
    s = lax.axis_index("s")
    base = (c * NS + s) * NCHW
    pltpu.sync_copy(zeros_hbm.at[pl.ds(s * RPT, RPT)], acc.at[pl.ds(s * RPT, RPT)])
    pltpu.sync_copy(src_hbm.at[pl.ds(base, NCHW)], sidx)
    pltpu.sync_copy(dst_hbm.at[pl.ds(base, NCHW)], didx)
    plsc.subcore_barrier()

    for b in range(NBUF):
        pltpu.async_copy(table_hbm.at[sidx.at[b]], bufs.at[b], gsems[b])

    def group(gi, carry):
        descs = []
        for b in range(NBUF):
            j = gi * NBUF + b
            pltpu.make_async_copy(table_hbm.at[sidx.at[j]], bufs.at[b],
                                  gsems[b]).wait()
            descs.append(pltpu.async_copy(bufs.at[b], acc.at[didx.at[j]],
                                          ssems[b], add=True))
        for b in range(NBUF):
            jn = (gi + 1) * NBUF + b
            descs[b].wait()
            pltpu.async_copy(table_hbm.at[sidx.at[jn]], bufs.at[b], gsems[b])
        return carry

    lax.fori_loop(0, NGRP - 1, group, 0)
    for b in range(NBUF):
        j = (NGRP - 1) * NBUF + b
        pltpu.make_async_copy(table_hbm.at[sidx.at[j]], bufs.at[b],
                              gsems[b]).wait()
        pltpu.async_copy(bufs.at[b], acc.at[didx.at[j]], ssems[b],
                         add=True).wait()
    plsc.subcore_barrier()
    pltpu.sync_copy(acc.at[pl.ds(s * RPT, RPT)], out_hbm.at[c, pl.ds(s * RPT, RPT)])


def _make_agg(interpret=False):
    return pl.kernel(
        _agg_body,
        out_type=jax.ShapeDtypeStruct((NC, NPAD, 128), jnp.float32),
        mesh=_mesh,
        scratch_types=[
            pltpu.VMEM((NCHW, CH), jnp.int32),
            pltpu.VMEM((NCHW, CH), jnp.int32),
            pltpu.VMEM((NBUF, CH, 128), jnp.float32),
            pltpu.VMEM_SHARED((NPAD, 128), jnp.float32),
        ] + [pltpu.SemaphoreType.DMA] * (2 * NBUF),
        interpret=interpret,
    )


_agg_kernel = _make_agg()


BR = 512
GRID = NPAD // BR


def _dvec(deg16):
    deg = deg16[0, :, 0] + deg16[1, :, 0] + 1.0
    return lax.rsqrt(deg)[:, None]


def _scale_body(deg16_ref, x_ref, y0_ref, y1_ref):
    d = _dvec(deg16_ref[...])
    y = x_ref[...] * d
    y0_ref[...] = y[:, :128]
    y1_ref[...] = y[:, 128:]


_scale_kernel = pl.pallas_call(
    _scale_body,
    grid=(GRID,),
    in_specs=[
        pl.BlockSpec((NC, BR, DEGW), lambda i: (0, i, 0)),
        pl.BlockSpec((BR, 256), lambda i: (i, 0)),
    ],
    out_specs=[
        pl.BlockSpec((BR, 128), lambda i: (i, 0)),
        pl.BlockSpec((BR, 128), lambda i: (i, 0)),
    ],
    out_shape=[
        jax.ShapeDtypeStruct((NPAD, 128), jnp.float32),
        jax.ShapeDtypeStruct((NPAD, 128), jnp.float32),
    ],
)


def _layer_body(s10_ref, s11_ref, deg16_ref, x_ref, w1_ref, b1_ref, w2_ref,
                y2_ref, z2_ref):
    d = _dvec(deg16_ref[...])
    s1 = jnp.concatenate(
        [s10_ref[0] + s10_ref[1], s11_ref[0] + s11_ref[1]], axis=1)
    z1 = d * s1 + (d * d) * x_ref[...]
    h = jnp.maximum(
        jnp.dot(z1, w1_ref[...], preferred_element_type=jnp.float32)
        + b1_ref[...], 0.0)
    z2 = jnp.dot(h, w2_ref[...], preferred_element_type=jnp.float32)
    z2_ref[...] = z2
    y2_ref[...] = d * z2


_layer_kernel = pl.pallas_call(
    _layer_body,
    grid=(GRID,),
    in_specs=[
        pl.BlockSpec((NC, BR, 128), lambda i: (0, i, 0)),
        pl.BlockSpec((NC, BR, 128), lambda i: (0, i, 0)),
        pl.BlockSpec((NC, BR, DEGW), lambda i: (0, i, 0)),
        pl.BlockSpec((BR, 256), lambda i: (i, 0)),
        pl.BlockSpec((256, 512), lambda i: (0, 0)),
        pl.BlockSpec((1, 512), lambda i: (0, 0)),
        pl.BlockSpec((512, 128), lambda i: (0, 0)),
    ],
    out_specs=[
        pl.BlockSpec((BR, 128), lambda i: (i, 0)),
        pl.BlockSpec((BR, 128), lambda i: (i, 0)),
    ],
    out_shape=[
        jax.ShapeDtypeStruct((NPAD, 128), jnp.float32),
        jax.ShapeDtypeStruct((NPAD, 128), jnp.float32),
    ],
)


def _out_body(s2_ref, deg16_ref, z2_ref, b2_ref, out_ref):
    d = _dvec(deg16_ref[...])
    out_ref[...] = (d * (s2_ref[0] + s2_ref[1])
                    + (d * d) * z2_ref[...] + b2_ref[...])


_out_kernel = pl.pallas_call(
    _out_body,
    grid=(GRID,),
    in_specs=[
        pl.BlockSpec((NC, BR, 128), lambda i: (0, i, 0)),
        pl.BlockSpec((NC, BR, DEGW), lambda i: (0, i, 0)),
        pl.BlockSpec((BR, 128), lambda i: (i, 0)),
        pl.BlockSpec((1, 128), lambda i: (0, 0)),
    ],
    out_specs=pl.BlockSpec((BR, 128), lambda i: (i, 0)),
    out_shape=jax.ShapeDtypeStruct((NPAD, 128), jnp.float32),
)


def kernel(x, edge_index, W1, b1, W2, b2):
    src = edge_index[0].astype(jnp.int32)
    dst = edge_index[1].astype(jnp.int32)
    fill_src = jnp.arange(EPAD - E, dtype=jnp.int32) % N
    fill_dst = jnp.full((EPAD - E,), N, jnp.int32)
    src_p = jnp.concatenate([src, fill_src]).reshape(EPAD // CH, CH)
    dst_p = jnp.concatenate([dst, fill_dst]).reshape(EPAD // CH, CH)
    x_p = jnp.zeros((NPAD, 256), jnp.float32).at[:N].set(x)
    zeros128 = jnp.zeros((NPAD, 128), jnp.float32)
    onesw = jnp.ones((CHD, DEGW), jnp.float32)

    deg16 = _deg_kernel(dst_p.reshape(EPAD // CHD, CHD), onesw, zeros128)
    y0, y1 = _scale_kernel(deg16, x_p)
    s10 = _agg_kernel(src_p, dst_p, y0, zeros128)
    s11 = _agg_kernel(src_p, dst_p, y1, zeros128)
    y2, z2 = _layer_kernel(s10, s11, deg16, x_p, W1, b1.reshape(1, -1), W2)
    s2 = _agg_kernel(src_p, dst_p, y2, zeros128)
    out = _out_kernel(s2, deg16, z2, b2.reshape(1, -1))
    return out[:N]
